# Initial kernel scaffold; baseline (speedup 1.0000x reference)
import jax
import jax.numpy as jnp
from jax.experimental import pallas as pl


def kernel(input, embedding, position_emb):
    raise NotImplementedError("write your pallas kernel here")


# speedup vs baseline: 1.2009x; 1.2009x over previous
import functools

import jax
import jax.numpy as jnp
from jax import lax
from jax.experimental import pallas as pl
from jax.experimental.pallas import tpu as pltpu
from jax.experimental.pallas import tpu_sc as plsc

_NUM = 1024 + 3 * 128 + 1
_DIM = 1024
_B = 4
_T = 2048

_NC = 2
_NS = 16
_NW = _NC * _NS
_PW = _T // _NW
_CH = 32
_STEPS = _PW // _CH


def _body(idx_hbm, table_hbm, pos_hbm, out_hbm, mask_hbm,
          pos_v, acc_v, idx_c, mask_c, sem):
    wid = lax.axis_index("s") * _NC + lax.axis_index("c")
    p0 = wid * _PW

    pltpu.sync_copy(pos_hbm.at[pl.ds(p0, _PW)], pos_v)

    for h in range(_STEPS):
        for b in range(_B):
            base = b * _T + p0 + h * _CH
            pltpu.sync_copy(idx_hbm.at[pl.ds(base, _CH)], idx_c)
            pltpu.async_copy(table_hbm.at[idx_c], acc_v, sem).wait()

            for k in range(_CH // 16):
                sl = pl.ds(k * 16, 16)
                ones = jnp.full((16,), 1, jnp.int32)
                zeros = jnp.full((16,), 0, jnp.int32)
                mask_c[sl] = jnp.where(idx_c[sl] == _NUM - 1, ones, zeros)
            pltpu.sync_copy(mask_c, mask_hbm.at[pl.ds(base, _CH)])

            def add_row(r, _):
                for j in range(_DIM // 16):
                    sl = pl.ds(j * 16, 16)
                    acc_v[r, sl] = acc_v[r, sl] + pos_v[h * _CH + r, sl]
                return 0

            lax.fori_loop(0, _CH, add_row, 0)

            pltpu.sync_copy(acc_v, out_hbm.at[pl.ds(base, _CH)])


@jax.jit
def _sc_lookup(idx, table, pos):
    mesh = plsc.VectorSubcoreMesh(core_axis_name="c", subcore_axis_name="s")
    return pl.kernel(
        _body,
        mesh=mesh,
        out_type=[
            jax.ShapeDtypeStruct((_B * _T, _DIM), jnp.float32),
            jax.ShapeDtypeStruct((_B * _T,), jnp.int32),
        ],
        scratch_types=[
            pltpu.VMEM((_PW, _DIM), jnp.float32),
            pltpu.VMEM((_CH, _DIM), jnp.float32),
            pltpu.VMEM((_CH,), jnp.int32),
            pltpu.VMEM((_CH,), jnp.int32),
            pltpu.SemaphoreType.DMA,
        ],
    )(idx, table, pos)


def kernel(input, embedding, position_emb):
    idx = input.reshape(_B * _T)
    pos = position_emb.reshape(_T, _DIM)
    emb_flat, mask_flat = _sc_lookup(idx, embedding, pos)
    return (emb_flat.reshape(_B, _T, _DIM), mask_flat.reshape(_B, _T))


# speedup vs baseline: 1.2512x; 1.0419x over previous
import functools

import jax
import jax.numpy as jnp
from jax import lax
from jax.experimental import pallas as pl
from jax.experimental.pallas import tpu as pltpu
from jax.experimental.pallas import tpu_sc as plsc

_NUM = 1024 + 3 * 128 + 1
_DIM = 1024
_B = 4
_T = 2048

_NC = 2
_NS = 16
_NW = _NC * _NS
_PW = _T // _NW
_CH = 16
_NSTEP = _PW // _CH * _B


def _body(idx_hbm, table_hbm, pos_hbm, out_hbm, mask_hbm,
          pos_v, idx_v, mask_v, acc0, acc1,
          sem_pos, sem_g0, sem_g1, sem_s0, sem_s1):
    wid = lax.axis_index("s") * _NC + lax.axis_index("c")
    p0 = wid * _PW

    pos_cp = pltpu.async_copy(pos_hbm.at[pl.ds(p0, _PW)], pos_v, sem_pos)

    for b in range(_B):
        pltpu.sync_copy(idx_hbm.at[pl.ds(b * _T + p0, _PW)], idx_v.at[b])
    for b in range(_B):
        for k in range(_PW // 16):
            sl = pl.ds(k * 16, 16)
            ones = jnp.full((16,), 1, jnp.int32)
            zeros = jnp.full((16,), 0, jnp.int32)
            mask_v[b, sl] = jnp.where(idx_v[b, sl] == _NUM - 1, ones, zeros)
    for b in range(_B):
        pltpu.sync_copy(mask_v.at[b], mask_hbm.at[pl.ds(b * _T + p0, _PW)])

    accs = (acc0, acc1)
    gsems = (sem_g0, sem_g1)
    ssems = (sem_s0, sem_s1)

    def gather(s):
        b, q = s % _B, s // _B
        return pltpu.async_copy(
            table_hbm.at[idx_v.at[b, pl.ds(q * _CH, _CH)]],
            accs[s % 2], gsems[s % 2])

    def store(s):
        b, q = s % _B, s // _B
        base = b * _T + p0 + q * _CH
        return pltpu.async_copy(
            accs[s % 2], out_hbm.at[pl.ds(base, _CH)], ssems[s % 2])

    g = {0: gather(0)}
    st = {}
    pos_cp.wait()
    for s in range(_NSTEP):
        g[s].wait()
        if s + 1 < _NSTEP:
            if s >= 1:
                st[s - 1].wait()
            g[s + 1] = gather(s + 1)
        q = s // _B
        acc = accs[s % 2]

        def add_row(r, _):
            for j in range(_DIM // 16):
                sl = pl.ds(j * 16, 16)
                acc[r, sl] = acc[r, sl] + pos_v[q * _CH + r, sl]
            return 0

        lax.fori_loop(0, _CH, add_row, 0)
        st[s] = store(s)
    st[_NSTEP - 2].wait()
    st[_NSTEP - 1].wait()


@jax.jit
def _sc_lookup(idx, table, pos):
    mesh = plsc.VectorSubcoreMesh(core_axis_name="c", subcore_axis_name="s")
    return pl.kernel(
        _body,
        mesh=mesh,
        out_type=[
            jax.ShapeDtypeStruct((_B * _T, _DIM), jnp.float32),
            jax.ShapeDtypeStruct((_B * _T,), jnp.int32),
        ],
        scratch_types=[
            pltpu.VMEM((_PW, _DIM), jnp.float32),
            pltpu.VMEM((_B, _PW), jnp.int32),
            pltpu.VMEM((_B, _PW), jnp.int32),
            pltpu.VMEM((_CH, _DIM), jnp.float32),
            pltpu.VMEM((_CH, _DIM), jnp.float32),
            pltpu.SemaphoreType.DMA,
            pltpu.SemaphoreType.DMA,
            pltpu.SemaphoreType.DMA,
            pltpu.SemaphoreType.DMA,
            pltpu.SemaphoreType.DMA,
        ],
    )(idx, table, pos)


def kernel(input, embedding, position_emb):
    idx = input.reshape(_B * _T)
    pos = position_emb.reshape(_T, _DIM)
    emb_flat, mask_flat = _sc_lookup(idx, embedding, pos)
    return (emb_flat.reshape(_B, _T, _DIM), mask_flat.reshape(_B, _T))
